# Initial kernel scaffold; baseline (speedup 1.0000x reference)
import jax
import jax.numpy as jnp
from jax.experimental import pallas as pl


def kernel(x, hidden, emb):
    raise NotImplementedError("write your pallas kernel here")


# speedup vs baseline: 1.5501x; 1.5501x over previous
import functools

import jax
import jax.numpy as jnp
from jax import lax
from jax.experimental import pallas as pl
from jax.experimental.pallas import tpu as pltpu
from jax.experimental.pallas import tpu_sc as plsc

VOCAB = 1000000
EMBED_DIM = 32
BATCH = 4096
SEQ = 200

_INFO = plsc.get_sparse_core_info()
NC, NS = _INFO.num_cores, _INFO.num_subcores
NW = NC * NS

N_ROWS = BATCH * SEQ
ROWS_PER_W = N_ROWS // NW
IDX_W = 128
ROWS_PER_CHUNK = 1024
GATHERS_PER_CHUNK = ROWS_PER_CHUNK // IDX_W
N_CHUNKS = ROWS_PER_W // ROWS_PER_CHUNK
IDX_ROWS = ROWS_PER_W // IDX_W


def _gather_body(emb_h, idx_h, out_h, idx_v, rows_v, gsem):
    wid = lax.axis_index("s") * NC + lax.axis_index("c")
    base = wid * ROWS_PER_W
    pltpu.sync_copy(idx_h.at[wid], idx_v)

    def chunk(c, carry):
        copies = [
            pltpu.async_copy(
                emb_h.at[idx_v.at[c * GATHERS_PER_CHUNK + j]],
                rows_v.at[pl.ds(j * IDX_W, IDX_W), :],
                gsem,
            )
            for j in range(GATHERS_PER_CHUNK)
        ]
        for cp in copies:
            cp.wait()
        pltpu.sync_copy(
            rows_v, out_h.at[pl.ds(base + c * ROWS_PER_CHUNK, ROWS_PER_CHUNK), :]
        )
        return carry

    lax.fori_loop(0, N_CHUNKS, chunk, 0)


@functools.partial(jax.jit, static_argnames=())
def kernel(x, hidden, emb):
    del hidden
    idx = jnp.transpose(x.astype(jnp.int32)).reshape(NW, IDX_ROWS, IDX_W)

    mesh = plsc.VectorSubcoreMesh(core_axis_name="c", subcore_axis_name="s")
    flat = pl.kernel(
        _gather_body,
        mesh=mesh,
        out_type=jax.ShapeDtypeStruct((N_ROWS, EMBED_DIM), jnp.float32),
        scratch_types=[
            pltpu.VMEM((IDX_ROWS, IDX_W), jnp.int32),
            pltpu.VMEM((ROWS_PER_CHUNK, EMBED_DIM), jnp.float32),
            pltpu.SemaphoreType.DMA,
        ],
        compiler_params=pltpu.CompilerParams(use_tc_tiling_on_sc=False),
    )(emb, idx)
    return flat.reshape(SEQ, BATCH, EMBED_DIM)


# speedup vs baseline: 1.5671x; 1.0110x over previous
import functools

import jax
import jax.numpy as jnp
from jax import lax
from jax.experimental import pallas as pl
from jax.experimental.pallas import tpu as pltpu
from jax.experimental.pallas import tpu_sc as plsc

VOCAB = 1000000
EMBED_DIM = 32
BATCH = 4096
SEQ = 200

_INFO = plsc.get_sparse_core_info()
NC, NS = _INFO.num_cores, _INFO.num_subcores
NW = NC * NS

N_ROWS = BATCH * SEQ
ROWS_PER_W = N_ROWS // NW
IDX_W = 128
GATHERS_PER_CHUNK = 10
ROWS_PER_CHUNK = IDX_W * GATHERS_PER_CHUNK
N_CHUNKS = ROWS_PER_W // ROWS_PER_CHUNK
N_PAIRS = N_CHUNKS // 2
IDX_ROWS = ROWS_PER_W // IDX_W


def _gather_body(emb_h, idx_h, out_h, idx_v, buf0, buf1, gsem0, gsem1,
                 wsem0, wsem1):
    wid = lax.axis_index("s") * NC + lax.axis_index("c")
    base = wid * ROWS_PER_W
    pltpu.sync_copy(idx_h.at[wid], idx_v)

    def fire(c, buf, sem):
        for j in range(GATHERS_PER_CHUNK):
            pltpu.async_copy(
                emb_h.at[idx_v.at[c * GATHERS_PER_CHUNK + j]],
                buf.at[pl.ds(j * IDX_W, IDX_W), :],
                sem,
            )

    def drain_gather(buf, sem):
        pltpu.make_async_copy(out_h.at[pl.ds(0, ROWS_PER_CHUNK), :], buf,
                              sem).wait()

    def writeback(c, buf, sem):
        pltpu.async_copy(
            buf, out_h.at[pl.ds(base + c * ROWS_PER_CHUNK, ROWS_PER_CHUNK), :],
            sem)

    def wait_writeback(buf, sem):
        pltpu.make_async_copy(buf, out_h.at[pl.ds(0, ROWS_PER_CHUNK), :],
                              sem).wait()

    fire(0, buf0, gsem0)
    fire(1, buf1, gsem1)

    def body(i, carry):
        c0 = 2 * i
        drain_gather(buf0, gsem0)
        writeback(c0, buf0, wsem0)
        drain_gather(buf1, gsem1)
        writeback(c0 + 1, buf1, wsem1)

        @pl.when(i < N_PAIRS - 1)
        def _():
            wait_writeback(buf0, wsem0)
            fire(c0 + 2, buf0, gsem0)
            wait_writeback(buf1, wsem1)
            fire(c0 + 3, buf1, gsem1)

        return carry

    lax.fori_loop(0, N_PAIRS, body, 0)
    wait_writeback(buf0, wsem0)
    wait_writeback(buf1, wsem1)


@functools.partial(jax.jit, static_argnames=())
def kernel(x, hidden, emb):
    del hidden
    idx = jnp.transpose(x.astype(jnp.int32)).reshape(NW, IDX_ROWS, IDX_W)

    mesh = plsc.VectorSubcoreMesh(core_axis_name="c", subcore_axis_name="s")
    flat = pl.kernel(
        _gather_body,
        mesh=mesh,
        out_type=jax.ShapeDtypeStruct((N_ROWS, EMBED_DIM), jnp.float32),
        scratch_types=[
            pltpu.VMEM((IDX_ROWS, IDX_W), jnp.int32),
            pltpu.VMEM((ROWS_PER_CHUNK, EMBED_DIM), jnp.float32),
            pltpu.VMEM((ROWS_PER_CHUNK, EMBED_DIM), jnp.float32),
            pltpu.SemaphoreType.DMA,
            pltpu.SemaphoreType.DMA,
            pltpu.SemaphoreType.DMA,
            pltpu.SemaphoreType.DMA,
        ],
        compiler_params=pltpu.CompilerParams(use_tc_tiling_on_sc=False),
    )(emb, idx)
    return flat.reshape(SEQ, BATCH, EMBED_DIM)
